# baseline (speedup 1.0000x reference)
import jax
import jax.numpy as jnp
from jax.experimental import pallas as pl
from jax.experimental.pallas import tpu as pltpu

IN_CH, N_HID, N_OUT = 32, 64, 64
ALPHA = 0.1
NV, NE, NF = 4096, 12288, 8192

BME = 384
BMV = 256
NEB = NE // BME
NVB = NV // BMV
BMF = NF // NEB
N_CELLS = NV + NE + NF
_VMEM_LIMIT = 65024 * 1024


def _leaky(x):
    return jnp.where(x >= 0, x, ALPHA * x)


def _bf(x):
    return x.astype(jnp.bfloat16)


def _edge_kernel(gl_ref, gr_ref, xe_ref, xf_ref, xfb_ref, weT_ref,
                 wf2eT_ref, wfT_ref, be_ref, bf2e_ref, bf_ref, out_ref):
    h = NF // 2
    m = (jnp.dot(_bf(gl_ref[...]), _bf(xf_ref[:h, :]),
                 preferred_element_type=jnp.float32)
         + jnp.dot(_bf(gr_ref[...]), _bf(xf_ref[h:, :]),
                   preferred_element_type=jnp.float32))
    pre = (jnp.dot(xe_ref[...], weT_ref[...], preferred_element_type=jnp.float32)
           + jnp.dot(m, wf2eT_ref[...], preferred_element_type=jnp.float32)
           + be_ref[...] + bf2e_ref[...])
    pre_f = jnp.dot(xfb_ref[...], wfT_ref[...],
                    preferred_element_type=jnp.float32) + bf_ref[...]
    out_ref[0, ...] = (jnp.sum(_leaky(pre), axis=0, keepdims=True)
                       + jnp.sum(_leaky(pre_f), axis=0, keepdims=True))


def _vertex_kernel(gl_ref, gr_ref, xv_ref, xe_ref, wvT_ref, we2vT_ref,
                   bv_ref, be2v_ref, out_ref):
    h = NE // 2
    m = (jnp.dot(_bf(gl_ref[...]), _bf(xe_ref[:h, :]),
                 preferred_element_type=jnp.float32)
         + jnp.dot(_bf(gr_ref[...]), _bf(xe_ref[h:, :]),
                   preferred_element_type=jnp.float32))
    pre = (jnp.dot(xv_ref[...], wvT_ref[...], preferred_element_type=jnp.float32)
           + jnp.dot(m, we2vT_ref[...], preferred_element_type=jnp.float32)
           + bv_ref[...] + be2v_ref[...])
    out_ref[0, ...] = jnp.sum(_leaky(pre), axis=0, keepdims=True)


def _combine_kernel(se_ref, sv_ref, wlinT_ref, blin_ref, out_ref):
    s = (jnp.sum(se_ref[...], axis=(0, 1)) + jnp.sum(sv_ref[...], axis=(0, 1)))[None, :]
    z = jnp.maximum(s * (1.0 / N_CELLS), 0.0)
    out_ref[...] = jnp.dot(z, wlinT_ref[...],
                           preferred_element_type=jnp.float32) + blin_ref[...]


@jax.jit
def kernel(xv, xe, xf, Ge2v, Gf2e, Wv, bv, We, be, Wf, bf,
           We2v, be2v, Wf2e, bf2e, Wlin, blin):
    xv2, xe2, xf2 = xv[0], xe[0], xf[0]
    row = lambda b: b.reshape(1, -1)
    const = lambda i: (0, 0)
    blk = lambda i: (i, 0)

    s_e = pl.pallas_call(
        _edge_kernel,
        grid=(NEB,),
        in_specs=[
            pl.BlockSpec((BME, NF // 2), lambda i: (i, 0)),
            pl.BlockSpec((BME, NF // 2), lambda i: (i, 1)),
            pl.BlockSpec((BME, IN_CH), blk),
            pl.BlockSpec((NF, IN_CH), const),
            pl.BlockSpec((BMF, IN_CH), blk),
            pl.BlockSpec((IN_CH, N_HID), const),
            pl.BlockSpec((IN_CH, N_HID), const),
            pl.BlockSpec((IN_CH, N_HID), const),
            pl.BlockSpec((1, N_HID), const),
            pl.BlockSpec((1, N_HID), const),
            pl.BlockSpec((1, N_HID), const),
        ],
        out_specs=pl.BlockSpec((1, 1, N_HID), lambda i: (i, 0, 0)),
        out_shape=jax.ShapeDtypeStruct((NEB, 1, N_HID), jnp.float32),
        compiler_params=pltpu.CompilerParams(
            dimension_semantics=("parallel",),
            vmem_limit_bytes=_VMEM_LIMIT),
    )(Gf2e, Gf2e, xe2, xf2, xf2, We.T, Wf2e.T, Wf.T,
      row(be), row(bf2e), row(bf))

    s_v = pl.pallas_call(
        _vertex_kernel,
        grid=(NVB,),
        in_specs=[
            pl.BlockSpec((BMV, NE // 2), lambda i: (i, 0)),
            pl.BlockSpec((BMV, NE // 2), lambda i: (i, 1)),
            pl.BlockSpec((BMV, IN_CH), blk),
            pl.BlockSpec((NE, IN_CH), const),
            pl.BlockSpec((IN_CH, N_HID), const),
            pl.BlockSpec((IN_CH, N_HID), const),
            pl.BlockSpec((1, N_HID), const),
            pl.BlockSpec((1, N_HID), const),
        ],
        out_specs=pl.BlockSpec((1, 1, N_HID), lambda i: (i, 0, 0)),
        out_shape=jax.ShapeDtypeStruct((NVB, 1, N_HID), jnp.float32),
        compiler_params=pltpu.CompilerParams(
            dimension_semantics=("parallel",),
            vmem_limit_bytes=_VMEM_LIMIT),
    )(Ge2v, Ge2v, xv2, xe2, Wv.T, We2v.T, row(bv), row(be2v))

    out = pl.pallas_call(
        _combine_kernel,
        out_shape=jax.ShapeDtypeStruct((1, N_OUT), jnp.float32),
    )(s_e, s_v, Wlin.T, row(blin))
    return out


# speedup vs baseline: 1.0019x; 1.0019x over previous
import jax
import jax.numpy as jnp
from jax.experimental import pallas as pl
from jax.experimental.pallas import tpu as pltpu

IN_CH, N_HID, N_OUT = 32, 64, 64
ALPHA = 0.1
NV, NE, NF = 4096, 12288, 8192

BME = 256
BMV = 256
NEB = NE // BME
NVB = NV // BMV
N_CELLS = NV + NE + NF
_VMEM_LIMIT = 65024 * 1024

_NT = (((1,), (1,)), ((), ()))


def _leaky(x):
    return jnp.where(x >= 0, x, ALPHA * x)


def _bf(x):
    return x.astype(jnp.bfloat16)


def _dot_nt(x, w):
    return jax.lax.dot_general(x, w, _NT, preferred_element_type=jnp.float32)


def _cxn_kernel(gf2e_ref, ge2v_ref, xv_ref, xe_ref, xf_ref,
                wv_ref, we_ref, wf_ref, we2v_ref, wf2e_ref,
                bv_ref, be_ref, bf_ref, be2v_ref, bf2e_ref,
                wlin_ref, blin_ref, out_ref, acc_ref):
    i = pl.program_id(0)

    @pl.when(i == 0)
    def _init():
        pre_f = _dot_nt(xf_ref[...], wf_ref[...]) + bf_ref[...]
        acc_ref[...] = jnp.sum(_leaky(pre_f), axis=0, keepdims=True)

    @pl.when(i < NEB)
    def _edge_phase():
        m = jnp.dot(_bf(gf2e_ref[...]), _bf(xf_ref[...]),
                    preferred_element_type=jnp.float32)
        xe_blk = xe_ref[pl.ds(i * BME, BME), :]
        pre = (_dot_nt(xe_blk, we_ref[...]) + _dot_nt(m, wf2e_ref[...])
               + be_ref[...] + bf2e_ref[...])
        acc_ref[...] += jnp.sum(_leaky(pre), axis=0, keepdims=True)

    @pl.when(i >= NEB)
    def _vertex_phase():
        j = i - NEB
        m = jnp.dot(_bf(ge2v_ref[...]), _bf(xe_ref[...]),
                    preferred_element_type=jnp.float32)
        xv_blk = xv_ref[pl.ds(j * BMV, BMV), :]
        pre = (_dot_nt(xv_blk, wv_ref[...]) + _dot_nt(m, we2v_ref[...])
               + bv_ref[...] + be2v_ref[...])
        acc_ref[...] += jnp.sum(_leaky(pre), axis=0, keepdims=True)

    @pl.when(i == NEB + NVB - 1)
    def _readout():
        z = jnp.maximum(acc_ref[...] * (1.0 / N_CELLS), 0.0)
        out_ref[...] = _dot_nt(z, wlin_ref[...]) + blin_ref[...]


@jax.jit
def kernel(xv, xe, xf, Ge2v, Gf2e, Wv, bv, We, be, Wf, bf,
           We2v, be2v, Wf2e, bf2e, Wlin, blin):
    row = lambda b: b.reshape(1, -1)
    const = lambda i: (0, 0)

    out = pl.pallas_call(
        _cxn_kernel,
        grid=(NEB + NVB,),
        in_specs=[
            pl.BlockSpec((BME, NF), lambda i: (jnp.minimum(i, NEB - 1), 0)),
            pl.BlockSpec((BMV, NE), lambda i: (jnp.maximum(i - NEB, 0), 0)),
            pl.BlockSpec((NV, IN_CH), const),
            pl.BlockSpec((NE, IN_CH), const),
            pl.BlockSpec((NF, IN_CH), const),
            pl.BlockSpec((N_HID, IN_CH), const),
            pl.BlockSpec((N_HID, IN_CH), const),
            pl.BlockSpec((N_HID, IN_CH), const),
            pl.BlockSpec((N_HID, IN_CH), const),
            pl.BlockSpec((N_HID, IN_CH), const),
            pl.BlockSpec((1, N_HID), const),
            pl.BlockSpec((1, N_HID), const),
            pl.BlockSpec((1, N_HID), const),
            pl.BlockSpec((1, N_HID), const),
            pl.BlockSpec((1, N_HID), const),
            pl.BlockSpec((N_OUT, N_HID), const),
            pl.BlockSpec((1, N_OUT), const),
        ],
        out_specs=pl.BlockSpec((1, N_OUT), const),
        out_shape=jax.ShapeDtypeStruct((1, N_OUT), jnp.float32),
        scratch_shapes=[pltpu.VMEM((1, N_HID), jnp.float32)],
        compiler_params=pltpu.CompilerParams(
            vmem_limit_bytes=_VMEM_LIMIT),
    )(Gf2e, Ge2v, xv[0], xe[0], xf[0],
      Wv, We, Wf, We2v, Wf2e,
      row(bv), row(be), row(bf), row(be2v), row(bf2e),
      Wlin, row(blin))
    return out


# speedup vs baseline: 1.0494x; 1.0474x over previous
import jax
import jax.numpy as jnp
from jax.experimental import pallas as pl
from jax.experimental.pallas import tpu as pltpu

IN_CH, N_HID, N_OUT = 32, 64, 64
ALPHA = 0.1
NV, NE, NF = 4096, 12288, 8192

BME = 256
BMV = 128
NEB = NE // BME
NVB = NV // BMV
NBUF = 3
N_CELLS = NV + NE + NF
_VMEM_LIMIT = 65024 * 1024


def _leaky(x):
    return jnp.where(x >= 0, x, ALPHA * x)


def _bf(x):
    return x.astype(jnp.bfloat16)


def _cxn_kernel(gf2e_hbm, ge2v_hbm, xv_ref, xe_ref, xf_ref,
                wvT_ref, weT_ref, wfT_ref, we2vT_ref, wf2eT_ref,
                bv_ref, be_ref, bf_ref, be2v_ref, bf2e_ref,
                wlinT_ref, blin_ref, out_ref,
                ebuf, vbuf, acc_ref, esem, vsem):

    def ecopy(idx, slot):
        return pltpu.make_async_copy(
            gf2e_hbm.at[pl.ds(idx * BME, BME), :], ebuf.at[slot],
            esem.at[slot])

    def vcopy(idx, slot):
        return pltpu.make_async_copy(
            ge2v_hbm.at[pl.ds(idx * BMV, BMV), :], vbuf.at[slot],
            vsem.at[slot])

    for b in range(NBUF):
        ecopy(b, b).start()
    for b in range(NBUF):
        vcopy(b, b).start()

    pre_f = jnp.dot(xf_ref[...], wfT_ref[...],
                    preferred_element_type=jnp.float32) + bf_ref[...]
    acc_ref[...] = jnp.sum(_leaky(pre_f), axis=0, keepdims=True)

    def e_step(i, carry):
        slot = jax.lax.rem(i, NBUF)
        ecopy(i, slot).wait()
        m = jnp.dot(_bf(ebuf[slot]), _bf(xf_ref[...]),
                    preferred_element_type=jnp.float32)
        xe_blk = xe_ref[pl.ds(i * BME, BME), :]
        pre = (jnp.dot(xe_blk, weT_ref[...], preferred_element_type=jnp.float32)
               + jnp.dot(m, wf2eT_ref[...], preferred_element_type=jnp.float32)
               + be_ref[...] + bf2e_ref[...])
        acc_ref[...] += jnp.sum(_leaky(pre), axis=0, keepdims=True)

        @pl.when(i + NBUF < NEB)
        def _():
            ecopy(i + NBUF, slot).start()
        return carry

    jax.lax.fori_loop(0, NEB, e_step, 0)

    def v_step(j, carry):
        slot = jax.lax.rem(j, NBUF)
        vcopy(j, slot).wait()
        m = jnp.dot(_bf(vbuf[slot]), _bf(xe_ref[...]),
                    preferred_element_type=jnp.float32)
        xv_blk = xv_ref[pl.ds(j * BMV, BMV), :]
        pre = (jnp.dot(xv_blk, wvT_ref[...], preferred_element_type=jnp.float32)
               + jnp.dot(m, we2vT_ref[...], preferred_element_type=jnp.float32)
               + bv_ref[...] + be2v_ref[...])
        acc_ref[...] += jnp.sum(_leaky(pre), axis=0, keepdims=True)

        @pl.when(j + NBUF < NVB)
        def _():
            vcopy(j + NBUF, slot).start()
        return carry

    jax.lax.fori_loop(0, NVB, v_step, 0)

    z = jnp.maximum(acc_ref[...] * (1.0 / N_CELLS), 0.0)
    out_ref[...] = jnp.dot(z, wlinT_ref[...],
                           preferred_element_type=jnp.float32) + blin_ref[...]


@jax.jit
def kernel(xv, xe, xf, Ge2v, Gf2e, Wv, bv, We, be, Wf, bf,
           We2v, be2v, Wf2e, bf2e, Wlin, blin):
    row = lambda b: b.reshape(1, -1)
    vspec = pl.BlockSpec(memory_space=pltpu.MemorySpace.VMEM)
    aspec = pl.BlockSpec(memory_space=pltpu.MemorySpace.HBM)

    out = pl.pallas_call(
        _cxn_kernel,
        in_specs=[aspec, aspec] + [vspec] * 15,
        out_specs=vspec,
        out_shape=jax.ShapeDtypeStruct((1, N_OUT), jnp.float32),
        scratch_shapes=[
            pltpu.VMEM((NBUF, BME, NF), jnp.float32),
            pltpu.VMEM((NBUF, BMV, NE), jnp.float32),
            pltpu.VMEM((1, N_HID), jnp.float32),
            pltpu.SemaphoreType.DMA((NBUF,)),
            pltpu.SemaphoreType.DMA((NBUF,)),
        ],
        compiler_params=pltpu.CompilerParams(
            vmem_limit_bytes=_VMEM_LIMIT),
    )(Gf2e, Ge2v, xv[0], xe[0], xf[0],
      Wv.T, We.T, Wf.T, We2v.T, Wf2e.T,
      row(bv), row(be), row(bf), row(be2v), row(bf2e),
      Wlin.T, row(blin))
    return out
